# baseline (speedup 1.0000x reference)
import jax
import jax.numpy as jnp
from jax import lax
from jax.experimental import pallas as pl
from jax.experimental.pallas import tpu as pltpu
from jax.experimental.pallas import tpu_sc as plsc

_NC = 2
_NS = 16
_L = 16


def _tf_uniform(k0, k1, B, K, tiny):
    u32 = jnp.uint32
    ib = jax.lax.broadcasted_iota(u32, (B, K), 0)
    ik = jax.lax.broadcasted_iota(u32, (B, K), 1)
    x1 = ib * u32(K) + ik
    ks2 = k0 ^ k1 ^ u32(0x1BD11BDA)
    x0 = jnp.full((B, K), k0, u32)
    x1 = x1 + k1

    def rnds(x0, x1, rots):
        for r in rots:
            x0 = x0 + x1
            x1 = (x1 << u32(r)) | (x1 >> u32(32 - r))
            x1 = x1 ^ x0
        return x0, x1

    r0 = (13, 15, 26, 6)
    r1 = (17, 29, 16, 24)
    x0, x1 = rnds(x0, x1, r0)
    x0 = x0 + k1
    x1 = x1 + ks2 + u32(1)
    x0, x1 = rnds(x0, x1, r1)
    x0 = x0 + ks2
    x1 = x1 + k0 + u32(2)
    x0, x1 = rnds(x0, x1, r0)
    x0 = x0 + k0
    x1 = x1 + k1 + u32(3)
    x0, x1 = rnds(x0, x1, r1)
    x0 = x0 + k1
    x1 = x1 + ks2 + u32(4)
    x0, x1 = rnds(x0, x1, r0)
    x0 = x0 + ks2
    x1 = x1 + k0 + u32(5)
    bits = x0 ^ x1
    fb = (bits >> u32(9)) | jnp.float32(1.0).view(jnp.uint32)
    fl = jax.lax.bitcast_convert_type(fb, jnp.float32) - jnp.float32(1.0)
    return jnp.maximum(jnp.float32(tiny),
                       fl * (jnp.float32(1.0) - jnp.float32(tiny))
                       + jnp.float32(tiny))


def _fwd_kernel(keys_ref, data_ref, mask_ref, init_ref, T_ref, p_out, prev):
    t = pl.program_id(0)
    nt = pl.num_programs(0)
    B = data_ref.shape[1]
    K = T_ref.shape[0]
    d = data_ref[0, :, :]
    m = mask_ref[0, :, :]
    iota = jax.lax.broadcasted_iota(jnp.int32, (B, K), 1)
    oh = (iota == d).astype(jnp.float32)
    masked = m == 1.0

    def emit(x):
        u = _tf_uniform(keys_ref[t, 0], keys_ref[t, 1], B, K,
                        jnp.finfo(jnp.float32).tiny)
        eps = jnp.where(t == nt - 1, jnp.float32(1e-20), jnp.float32(0.0))
        p_out[0, :, :] = (x + eps) * (-1.0 / jnp.log(u))
        prev[:, :] = x

    @pl.when(t == 0)
    def _first():
        x = jnp.where(masked, oh, init_ref[0, :][None, :])
        s = jnp.sum(x, axis=1, keepdims=True)
        emit(x / (s + 1e-8))

    @pl.when(t > 0)
    def _step():
        mm = jnp.dot(prev[:, :], T_ref[:, :],
                     preferred_element_type=jnp.float32)
        x = jnp.where(masked, oh, mm)
        s = jnp.sum(x, axis=1, keepdims=True)
        emit(x / s)


def _make_bwd_sc(B, S, K):
    BPW = B // (_NC * _NS)
    NG = BPW // _L

    def body(p_hbm, tt_hbm, mask_hbm, data_hbm, out_hbm,
             pbuf, trow_v, mask_v, data_v, out_v, idx_v, psem, gsem):
        w = lax.axis_index("s") * _NC + lax.axis_index("c")
        base = w * BPW
        lane = lax.iota(jnp.int32, _L)

        pltpu.sync_copy(mask_hbm.at[w], mask_v)
        pltpu.sync_copy(data_hbm.at[w], data_v)

        def sample_group(par, g, is_last):
            row_idx = lane + g * _L

            def at_k(k, vmax, vidx):
                kv = jnp.full((_L,), k, jnp.int32)
                pv = plsc.load_gather(pbuf.at[par], [row_idx, kv])
                if is_last:
                    wv = pv
                else:
                    tr = plsc.load_gather(trow_v, [row_idx, kv])
                    wv = pv * (tr + 0.001)
                gt = wv > vmax
                return (jnp.where(gt, wv, vmax),
                        jnp.where(gt, kv, vidx))

            def body4(kk, carry):
                vmax, vidx = carry
                for q in range(4):
                    vmax, vidx = at_k(kk * 4 + q, vmax, vidx)
                return vmax, vidx

            vmax = jnp.full((_L,), -1.0, jnp.float32)
            vidx = jnp.full((_L,), 0, jnp.int32)
            n4 = K // 4
            vmax, vidx = lax.fori_loop(0, n4, body4, (vmax, vidx))
            for k in range(n4 * 4, K):
                vmax, vidx = at_k(k, vmax, vidx)
            return vidx

        def chain_pass(par, pos, is_last):
            outs = []
            for g in range(NG):
                mvec = mask_v[pos, pl.ds(g * _L, _L)]
                dvec = data_v[pos, pl.ds(g * _L, _L)]
                samp = sample_group(par, g, is_last)
                if is_last:
                    ov = jnp.where(mvec == 1.0, samp, dvec)
                else:
                    ov = jnp.where(mvec == 1.0, dvec, samp)
                out_v[pos, pl.ds(g * _L, _L)] = ov
                outs.append(ov)
            return outs

        pltpu.sync_copy(p_hbm.at[S - 1, pl.ds(base, BPW), :], pbuf.at[0])
        pltpu.async_copy(p_hbm.at[S - 2, pl.ds(base, BPW), :], pbuf.at[1],
                         psem)
        i0, i1 = chain_pass(0, S - 1, True)

        def step(t, carry):
            i0, i1 = carry
            pos = S - 1 - t
            par = t & 1
            idx_v[pl.ds(0, _L)] = i0
            idx_v[pl.ds(_L, _L)] = i1
            gather = pltpu.async_copy(tt_hbm.at[idx_v], trow_v, gsem)
            pltpu.make_async_copy(
                p_hbm.at[pos, pl.ds(base, BPW), :], pbuf.at[par], psem
            ).wait()
            nxt = jnp.maximum(pos - 1, 0)
            pltpu.async_copy(p_hbm.at[nxt, pl.ds(base, BPW), :],
                             pbuf.at[1 - par], psem)
            gather.wait()
            i0, i1 = chain_pass(par, pos, False)
            return i0, i1

        lax.fori_loop(1, S, step, (i0, i1))
        pltpu.make_async_copy(
            p_hbm.at[0, pl.ds(base, BPW), :], pbuf.at[S & 1], psem
        ).wait()
        pltpu.sync_copy(out_v, out_hbm.at[w])

    return pl.kernel(
        body,
        out_type=jax.ShapeDtypeStruct((_NC * _NS, S, BPW), jnp.int32),
        mesh=plsc.VectorSubcoreMesh(core_axis_name="c", subcore_axis_name="s"),
        scratch_types=[
            pltpu.VMEM((2, BPW, K), jnp.float32),
            pltpu.VMEM((BPW, K), jnp.float32),
            pltpu.VMEM((S, BPW), jnp.float32),
            pltpu.VMEM((S, BPW), jnp.int32),
            pltpu.VMEM((S, BPW), jnp.int32),
            pltpu.VMEM((BPW,), jnp.int32),
            pltpu.SemaphoreType.DMA,
            pltpu.SemaphoreType.DMA,
        ],
        compiler_params=pltpu.CompilerParams(
            use_tc_tiling_on_sc=False, needs_layout_passes=False),
    )


def kernel(data, masks, init_probability, transition_probability, nb_imputation):
    B, S = data.shape
    K = init_probability.shape[0]
    f32 = jnp.float32
    NW = _NC * _NS
    BPW = B // NW

    data_s = jnp.transpose(data, (1, 0))[:, :, None]
    masks_s = jnp.transpose(masks, (1, 0))[:, :, None]
    init2 = init_probability[None, :]

    skey = jax.random.key(42)
    klast, kloop = jax.random.split(skey)
    keys = [jax.random.key_data(klast)]
    key = kloop
    for _ in range(S - 1):
        key, sk = jax.random.split(key)
        keys.append(jax.random.key_data(sk))
    keys = jnp.stack(list(reversed(keys))).astype(jnp.uint32)

    P = pl.pallas_call(
        _fwd_kernel,
        grid=(S,),
        in_specs=[
            pl.BlockSpec(memory_space=pltpu.SMEM),
            pl.BlockSpec((1, B, 1), lambda t: (t, 0, 0)),
            pl.BlockSpec((1, B, 1), lambda t: (t, 0, 0)),
            pl.BlockSpec((1, K), lambda t: (0, 0)),
            pl.BlockSpec((K, K), lambda t: (0, 0)),
        ],
        out_specs=pl.BlockSpec((1, B, K), lambda t: (t, 0, 0)),
        out_shape=jax.ShapeDtypeStruct((S, B, K), f32),
        scratch_shapes=[pltpu.VMEM((B, K), f32)],
        compiler_params=pltpu.CompilerParams(
            dimension_semantics=("arbitrary",)),
    )(keys, data_s, masks_s, init2, transition_probability)

    Tt = jnp.transpose(transition_probability, (1, 0))
    mask_w = jnp.transpose(masks_s[:, :, 0].reshape(S, NW, BPW), (1, 0, 2))
    data_w = jnp.transpose(data_s[:, :, 0].reshape(S, NW, BPW), (1, 0, 2))

    out_w = _make_bwd_sc(B, S, K)(P, Tt, mask_w, data_w)

    out = jnp.transpose(out_w, (1, 0, 2)).reshape(S, B)
    return jnp.transpose(out, (1, 0))[:, None, :]


# speedup vs baseline: 1.4817x; 1.4817x over previous
import jax
import jax.numpy as jnp
from jax import lax
from jax.experimental import pallas as pl
from jax.experimental.pallas import tpu as pltpu
from jax.experimental.pallas import tpu_sc as plsc

_NC = 2
_NS = 16
_L = 16


def _fwd_kernel(data_ref, mask_ref, init_ref, T_ref, msg_out, prev):
    t = pl.program_id(0)
    B = data_ref.shape[1]
    K = T_ref.shape[0]
    d = data_ref[0, :, :]
    m = mask_ref[0, :, :]
    iota = jax.lax.broadcasted_iota(jnp.int32, (B, K), 1)
    oh = (iota == d).astype(jnp.float32)
    masked = m == 1.0

    @pl.when(t == 0)
    def _first():
        x = jnp.where(masked, oh, init_ref[0, :][None, :])
        s = jnp.sum(x, axis=1, keepdims=True)
        x = x / (s + 1e-8)
        msg_out[0, :, :] = x
        prev[:, :] = x

    @pl.when(t > 0)
    def _step():
        mm = jnp.dot(prev[:, :], T_ref[:, :],
                     preferred_element_type=jnp.float32)
        x = jnp.where(masked, oh, mm)
        s = jnp.sum(x, axis=1, keepdims=True)
        x = x / s
        msg_out[0, :, :] = x
        prev[:, :] = x


def _make_bwd_sc(B, S, K, CS, lo, first):
    BPW = B // (_NC * _NS)
    NG = BPW // _L

    def body(p_hbm, tt_hbm, mask_hbm, data_hbm, cin_hbm,
             out_hbm, cout_hbm,
             pbuf, trow_v, mask_v, data_v, out_v, idx_v, psem, gsem):
        w = lax.axis_index("s") * _NC + lax.axis_index("c")
        base = w * BPW
        lane = lax.iota(jnp.int32, _L)

        pltpu.sync_copy(mask_hbm.at[w], mask_v)
        pltpu.sync_copy(data_hbm.at[w], data_v)

        def sample_group(par, g, is_last):
            row_idx = lane + g * _L

            def at_k(k, vmax, vidx):
                kv = jnp.full((_L,), k, jnp.int32)
                pv = plsc.load_gather(pbuf.at[par], [row_idx, kv])
                if is_last:
                    wv = pv
                else:
                    tr = plsc.load_gather(trow_v, [row_idx, kv])
                    wv = pv * (tr + 0.001)
                gt = wv > vmax
                return (jnp.where(gt, wv, vmax),
                        jnp.where(gt, kv, vidx))

            def body4(kk, carry):
                vmax, vidx = carry
                for q in range(4):
                    vmax, vidx = at_k(kk * 4 + q, vmax, vidx)
                return vmax, vidx

            vmax = jnp.full((_L,), -1.0, jnp.float32)
            vidx = jnp.full((_L,), 0, jnp.int32)
            n4 = K // 4
            vmax, vidx = lax.fori_loop(0, n4, body4, (vmax, vidx))
            for k in range(n4 * 4, K):
                vmax, vidx = at_k(k, vmax, vidx)
            return vidx

        def chain_pass(par, p_loc, is_last):
            outs = []
            for g in range(NG):
                mvec = mask_v[lo + p_loc, pl.ds(g * _L, _L)]
                dvec = data_v[lo + p_loc, pl.ds(g * _L, _L)]
                samp = sample_group(par, g, is_last)
                if is_last:
                    ov = jnp.where(mvec == 1.0, samp, dvec)
                else:
                    ov = jnp.where(mvec == 1.0, dvec, samp)
                out_v[p_loc, pl.ds(g * _L, _L)] = ov
                outs.append(ov)
            return outs

        pltpu.sync_copy(p_hbm.at[CS - 1, pl.ds(base, BPW), :], pbuf.at[0])
        pltpu.async_copy(p_hbm.at[CS - 2, pl.ds(base, BPW), :], pbuf.at[1],
                         psem)
        if first:
            i0, i1 = chain_pass(0, CS - 1, True)
        else:
            pltpu.sync_copy(cin_hbm.at[w], idx_v)
            i0 = idx_v[pl.ds(0, _L)]
            i1 = idx_v[pl.ds(_L, _L)]
            gather = pltpu.async_copy(tt_hbm.at[idx_v], trow_v, gsem)
            gather.wait()
            i0, i1 = chain_pass(0, CS - 1, False)

        def step(j, carry):
            i0, i1 = carry
            p_loc = CS - 1 - j
            par = j & 1
            idx_v[pl.ds(0, _L)] = i0
            idx_v[pl.ds(_L, _L)] = i1
            gather = pltpu.async_copy(tt_hbm.at[idx_v], trow_v, gsem)
            pltpu.make_async_copy(
                p_hbm.at[p_loc, pl.ds(base, BPW), :], pbuf.at[par], psem
            ).wait()
            nxt = jnp.maximum(p_loc - 1, 0)
            pltpu.async_copy(p_hbm.at[nxt, pl.ds(base, BPW), :],
                             pbuf.at[1 - par], psem)
            gather.wait()
            i0, i1 = chain_pass(par, p_loc, False)
            return i0, i1

        i0, i1 = lax.fori_loop(1, CS, step, (i0, i1))
        pltpu.make_async_copy(
            p_hbm.at[0, pl.ds(base, BPW), :], pbuf.at[CS & 1], psem
        ).wait()
        idx_v[pl.ds(0, _L)] = i0
        idx_v[pl.ds(_L, _L)] = i1
        pltpu.sync_copy(out_v, out_hbm.at[w])
        pltpu.sync_copy(idx_v, cout_hbm.at[w])

    return pl.kernel(
        body,
        out_type=(
            jax.ShapeDtypeStruct((_NC * _NS, CS, BPW), jnp.int32),
            jax.ShapeDtypeStruct((_NC * _NS, BPW), jnp.int32),
        ),
        mesh=plsc.VectorSubcoreMesh(core_axis_name="c", subcore_axis_name="s"),
        scratch_types=[
            pltpu.VMEM((2, BPW, K), jnp.float32),
            pltpu.VMEM((BPW, K), jnp.float32),
            pltpu.VMEM((S, BPW), jnp.float32),
            pltpu.VMEM((S, BPW), jnp.int32),
            pltpu.VMEM((CS, BPW), jnp.int32),
            pltpu.VMEM((BPW,), jnp.int32),
            pltpu.SemaphoreType.DMA,
            pltpu.SemaphoreType.DMA,
        ],
        compiler_params=pltpu.CompilerParams(
            use_tc_tiling_on_sc=False, needs_layout_passes=False),
    )


def kernel(data, masks, init_probability, transition_probability, nb_imputation):
    B, S = data.shape
    K = init_probability.shape[0]
    f32 = jnp.float32
    NW = _NC * _NS
    BPW = B // NW

    data_s = jnp.transpose(data, (1, 0))[:, :, None]
    masks_s = jnp.transpose(masks, (1, 0))[:, :, None]
    init2 = init_probability[None, :]

    messages = pl.pallas_call(
        _fwd_kernel,
        grid=(S,),
        in_specs=[
            pl.BlockSpec((1, B, 1), lambda t: (t, 0, 0)),
            pl.BlockSpec((1, B, 1), lambda t: (t, 0, 0)),
            pl.BlockSpec((1, K), lambda t: (0, 0)),
            pl.BlockSpec((K, K), lambda t: (0, 0)),
        ],
        out_specs=pl.BlockSpec((1, B, K), lambda t: (t, 0, 0)),
        out_shape=jax.ShapeDtypeStruct((S, B, K), f32),
        scratch_shapes=[pltpu.VMEM((B, K), f32)],
        compiler_params=pltpu.CompilerParams(
            dimension_semantics=("arbitrary",)),
    )(data_s, masks_s, init2, transition_probability)

    tiny = jnp.finfo(f32).tiny
    skey = jax.random.key(42)
    klast, kloop = jax.random.split(skey)
    us = [jax.random.uniform(klast, (1, B, K), f32, minval=tiny, maxval=1.)
          .reshape(1, B, K)]
    key = kloop
    for _ in range(S - 1):
        key, sk = jax.random.split(key)
        us.append(jax.random.uniform(sk, (B, 1, K), f32, minval=tiny, maxval=1.)
                  .reshape(1, B, K))
    us = list(reversed(us))

    Tt = jnp.transpose(transition_probability, (1, 0))
    mask_w = jnp.transpose(masks_s[:, :, 0].reshape(S, NW, BPW), (1, 0, 2))
    data_w = jnp.transpose(data_s[:, :, 0].reshape(S, NW, BPW), (1, 0, 2))

    CS = 5
    chunks = [(S - CS * (c + 1), c == 0) for c in range((S + CS - 1) // CS)]
    carry = jnp.zeros((NW, BPW), jnp.int32)
    outs = {}
    for lo, first in chunks:
        U_c = jnp.concatenate(us[lo:lo + CS], axis=0)
        eps = jnp.zeros((CS, 1, 1), f32)
        if first:
            eps = eps.at[CS - 1].set(1e-20)
        P_c = (lax.slice_in_dim(messages, lo, lo + CS, axis=0) + eps) * (
            -1.0 / jnp.log(U_c))
        out_c, carry = _make_bwd_sc(B, S, K, CS, lo, first)(
            P_c, Tt, mask_w, data_w, carry)
        outs[lo] = out_c

    out_w = jnp.concatenate([outs[lo] for lo, _ in sorted(chunks)], axis=1)
    out = jnp.transpose(out_w, (1, 0, 2)).reshape(S, B)
    return jnp.transpose(out, (1, 0))[:, None, :]
